# baseline (speedup 1.0000x reference)
import jax
import jax.numpy as jnp
from jax import lax
from jax.experimental import pallas as pl
from jax.experimental.pallas import tpu as pltpu
from jax.experimental.pallas import tpu_sc as plsc

_H = 256
_W = 256
_RADIUS_RX = 1.0
_SCALE_DIS = 1.5
_WAVELENGTH = 0.1

_N = 500000
_COLS = 512
_PAD_ROWS = 1024
_NPADC = _PAD_ROWS * _COLS - _N
_BLK = 256
_GRID = _PAD_ROWS // _BLK

_NW = 32
_CROWS = _PAD_ROWS // _NW
_OC = 512
_PR = 512
_PC = 128
_PIXV = _PR * _PC


_BLKC = _BLK * _COLS


def _tc_body(rx_ref, tx_ref, m_ref, cov_ref, sig_ref, att_ref, rad_ref,
             idx_ref, re_ref, im_ref):
    i = pl.program_id(0)
    mx = m_ref[0].reshape(_BLK, _COLS)
    my = m_ref[1].reshape(_BLK, _COLS)
    mz = m_ref[2].reshape(_BLK, _COLS)
    dx = mx - rx_ref[0]
    dy = my - rx_ref[1]
    dz = mz - rx_ref[2]
    d_rx = jnp.sqrt(dx * dx + dy * dy + dz * dz)
    ex = mx - tx_ref[0]
    ey = my - tx_ref[1]
    ez = mz - tx_ref[2]
    d_tx = jnp.sqrt(ex * ex + ey * ey + ez * ez)
    total = d_rx + d_tx

    att = att_ref[0].reshape(_BLK, _COLS)
    amp = jnp.exp(-att * total) / jnp.maximum(total, 1e-6)
    phase = 2.0 * jnp.pi * total / _WAVELENGTH
    c = jnp.cos(phase)
    s = jnp.sin(phase)
    sr = sig_ref[0].reshape(_BLK, _COLS)
    si = sig_ref[1].reshape(_BLK, _COLS)
    re = amp * (sr * c - si * s)
    im = amp * (sr * s + si * c)

    c2 = (cov_ref[0].reshape(_BLK, _COLS) ** 2
          + cov_ref[1].reshape(_BLK, _COLS) ** 2
          + cov_ref[2].reshape(_BLK, _COLS) ** 2
          + cov_ref[3].reshape(_BLK, _COLS) ** 2
          + cov_ref[4].reshape(_BLK, _COLS) ** 2
          + cov_ref[5].reshape(_BLK, _COLS) ** 2)
    rad = rad_ref[0].reshape(_BLK, _COLS)
    weight = jnp.exp(-0.5 * c2 / (rad * rad + 1e-6))
    scale = jnp.where(d_rx > _RADIUS_RX * _SCALE_DIS, weight, 0.0)

    az = jnp.arctan2(dy, dx)
    t = jnp.clip(dz / jnp.maximum(d_rx, 1e-6), -1.0, 1.0)
    el = 2.0 * jnp.arctan2(t, 1.0 + jnp.sqrt(1.0 - t * t))
    u = jnp.clip(((az + jnp.pi) / (2.0 * jnp.pi) * _W).astype(jnp.int32),
                 0, _W - 1)
    v = jnp.clip(((el + jnp.pi / 2.0) / jnp.pi * _H).astype(jnp.int32),
                 0, _H - 1)

    g = (i * _BLKC + 512 * lax.broadcasted_iota(jnp.int32, (_BLK, _COLS), 0)
         + lax.broadcasted_iota(jnp.int32, (_BLK, _COLS), 1))
    valid = g < _N
    idx_ref[...] = jnp.where(valid, v * _W + u, 0)
    re_ref[...] = jnp.where(valid, re * scale, 0.0)
    im_ref[...] = jnp.where(valid, im * scale, 0.0)


def _in_map(i):
    return (0, i)


def _in_map2(i):
    return (i, 0)


_tc_call = pl.pallas_call(
    _tc_body,
    grid=(_GRID,),
    in_specs=[
        pl.BlockSpec(memory_space=pltpu.SMEM),
        pl.BlockSpec(memory_space=pltpu.SMEM),
        pl.BlockSpec((3, _BLKC), _in_map),
        pl.BlockSpec((6, _BLKC), _in_map),
        pl.BlockSpec((2, _BLKC), _in_map),
        pl.BlockSpec((1, _BLKC), _in_map),
        pl.BlockSpec((1, _BLKC), _in_map),
    ],
    out_specs=[
        pl.BlockSpec((_BLK, _OC), lambda i: (i, 0)),
        pl.BlockSpec((_BLK, _OC), lambda i: (i, 0)),
        pl.BlockSpec((_BLK, _OC), lambda i: (i, 0)),
    ],
    out_shape=[
        jax.ShapeDtypeStruct((_PAD_ROWS, _OC), jnp.int32),
        jax.ShapeDtypeStruct((_PAD_ROWS, _OC), jnp.float32),
        jax.ShapeDtypeStruct((_PAD_ROWS, _OC), jnp.float32),
    ],
    compiler_params=pltpu.CompilerParams(
        allow_input_fusion=[False, False, True, True, True, True, True]),
)


def _sc_body(idx_hbm, re_hbm, im_hbm, out_hbm,
             idx_v, re_v, im_v, plane_v, rows_v, accum_a, accum_b,
             sem_i, sem_r, sem_m):
    c = lax.axis_index("c")
    s = lax.axis_index("s")
    wid = c * 16 + s
    base = wid * _CROWS

    cp_i = pltpu.async_copy(idx_hbm.at[pl.ds(base, _CROWS)], idx_v, sem_i)
    cp_r = pltpu.async_copy(re_hbm.at[pl.ds(base, _CROWS)], re_v, sem_r)
    cp_m = pltpu.async_copy(im_hbm.at[pl.ds(base, _CROWS)], im_v, sem_m)

    zeros16f = jnp.zeros((16,), jnp.float32)

    def zero_plane():
        @plsc.parallel_loop(0, _PR * 8, unroll=8)
        def _(k):
            plane_v[k >> 3, pl.ds((k & 7) * 16, 16)] = zeros16f

    zero_plane()

    @plsc.parallel_loop(0, _PR // 16, unroll=4)
    def _(k):
        rows_v[pl.ds(k * 16, 16)] = lax.iota(jnp.int32, 16) + k * 16

    @pl.when(s == 0)
    def _():
        pltpu.sync_copy(plane_v, accum_a)
        pltpu.sync_copy(plane_v, accum_b)

    plsc.subcore_barrier()
    cp_i.wait()
    cp_r.wait()

    def scatter_pass(val_v):
        @plsc.parallel_loop(0, _CROWS * (_OC // 16), unroll=8)
        def _(t):
            r = t >> 5
            q = lax.bitwise_and(t, 31) * 16
            iv = idx_v[r, pl.ds(q, 16)]
            hi = lax.shift_right_logical(iv, 7)
            lo = lax.bitwise_and(iv, 127)
            rv = val_v[r, pl.ds(q, 16)]
            plsc.addupdate_scatter(plane_v, [hi, lo], rv)

    scatter_pass(re_v)
    pltpu.sync_copy(plane_v, accum_a.at[rows_v], add=True)

    zero_plane()
    cp_m.wait()
    scatter_pass(im_v)
    pltpu.sync_copy(plane_v, accum_b.at[rows_v], add=True)

    plsc.subcore_barrier()

    orow = s * (_PR // 16)
    pltpu.sync_copy(accum_a.at[pl.ds(orow, _PR // 16)],
                    out_hbm.at[c, 0, pl.ds(orow, _PR // 16)])
    pltpu.sync_copy(accum_b.at[pl.ds(orow, _PR // 16)],
                    out_hbm.at[c, 1, pl.ds(orow, _PR // 16)])


_sc_call_cache = []


def _sc_call(idx, re, im):
    if not _sc_call_cache:
        _sc_call_cache.append(pl.kernel(
            _sc_body,
            out_type=jax.ShapeDtypeStruct((2, 2, _PR, _PC), jnp.float32),
            mesh=plsc.VectorSubcoreMesh(core_axis_name="c",
                                        subcore_axis_name="s"),
            compiler_params=pltpu.CompilerParams(needs_layout_passes=False),
            scratch_types=[
                pltpu.VMEM((_CROWS, _OC), jnp.int32),
                pltpu.VMEM((_CROWS, _OC), jnp.float32),
                pltpu.VMEM((_CROWS, _OC), jnp.float32),
                pltpu.VMEM((_PR, _PC), jnp.float32),
                pltpu.VMEM((_PR,), jnp.int32),
                pltpu.VMEM_SHARED((_PR, _PC), jnp.float32),
                pltpu.VMEM_SHARED((_PR, _PC), jnp.float32),
                pltpu.SemaphoreType.DMA,
                pltpu.SemaphoreType.DMA,
                pltpu.SemaphoreType.DMA,
            ],
        ))
    return _sc_call_cache[0](idx, re, im)


def kernel(means_3d, cov3d_precomp, signal_precomp, attenuation, gaus_radii,
           rx_pos, tx_pos, bg):
    mt = means_3d.T
    ct = cov3d_precomp.T
    st = signal_precomp.T
    att = attenuation[None, :]
    rad = gaus_radii[None, :]
    idx, re, im = _tc_call(rx_pos, tx_pos, mt, ct, st, att, rad)
    partial = _sc_call(idx, re, im)
    planes = partial[0] + partial[1]
    img = planes.reshape(2, _H, _W).transpose(1, 2, 0)
    return img + bg[None, None, :]


# speedup vs baseline: 1.0760x; 1.0760x over previous
import jax
import jax.numpy as jnp
from jax import lax
from jax.experimental import pallas as pl
from jax.experimental.pallas import tpu as pltpu
from jax.experimental.pallas import tpu_sc as plsc

_H = 256
_W = 256
_RADIUS_RX = 1.0
_SCALE_DIS = 1.5
_WAVELENGTH = 0.1

_N = 500000
_COLS = 512
_PAD_ROWS = 1024
_NPADC = _PAD_ROWS * _COLS - _N
_BLK = 64
_GRID = _PAD_ROWS // _BLK

_NW = 32
_CROWS = _PAD_ROWS // _NW
_OC = 512
_PR = 512
_PC = 128
_PIXV = _PR * _PC


_BLKC = _BLK * _COLS


def _tc_body(rx_ref, tx_ref, m_ref, cov_ref, sig_ref, att_ref, rad_ref,
             idx_ref, re_ref, im_ref):
    i = pl.program_id(0)
    mx = m_ref[0].reshape(_BLK, _COLS)
    my = m_ref[1].reshape(_BLK, _COLS)
    mz = m_ref[2].reshape(_BLK, _COLS)
    dx = mx - rx_ref[0]
    dy = my - rx_ref[1]
    dz = mz - rx_ref[2]
    d_rx = jnp.sqrt(dx * dx + dy * dy + dz * dz)
    ex = mx - tx_ref[0]
    ey = my - tx_ref[1]
    ez = mz - tx_ref[2]
    d_tx = jnp.sqrt(ex * ex + ey * ey + ez * ez)
    total = d_rx + d_tx

    att = att_ref[0].reshape(_BLK, _COLS)
    amp = jnp.exp(-att * total) / jnp.maximum(total, 1e-6)
    phase = 2.0 * jnp.pi * total / _WAVELENGTH
    c = jnp.cos(phase)
    s = jnp.sin(phase)
    sr = sig_ref[0].reshape(_BLK, _COLS)
    si = sig_ref[1].reshape(_BLK, _COLS)
    re = amp * (sr * c - si * s)
    im = amp * (sr * s + si * c)

    c2 = (cov_ref[0].reshape(_BLK, _COLS) ** 2
          + cov_ref[1].reshape(_BLK, _COLS) ** 2
          + cov_ref[2].reshape(_BLK, _COLS) ** 2
          + cov_ref[3].reshape(_BLK, _COLS) ** 2
          + cov_ref[4].reshape(_BLK, _COLS) ** 2
          + cov_ref[5].reshape(_BLK, _COLS) ** 2)
    rad = rad_ref[0].reshape(_BLK, _COLS)
    weight = jnp.exp(-0.5 * c2 / (rad * rad + 1e-6))
    scale = jnp.where(d_rx > _RADIUS_RX * _SCALE_DIS, weight, 0.0)

    az = jnp.arctan2(dy, dx)
    t = jnp.clip(dz / jnp.maximum(d_rx, 1e-6), -1.0, 1.0)
    el = 2.0 * jnp.arctan2(t, 1.0 + jnp.sqrt(1.0 - t * t))
    u = jnp.clip(((az + jnp.pi) / (2.0 * jnp.pi) * _W).astype(jnp.int32),
                 0, _W - 1)
    v = jnp.clip(((el + jnp.pi / 2.0) / jnp.pi * _H).astype(jnp.int32),
                 0, _H - 1)

    g = (i * _BLKC + 512 * lax.broadcasted_iota(jnp.int32, (_BLK, _COLS), 0)
         + lax.broadcasted_iota(jnp.int32, (_BLK, _COLS), 1))
    valid = g < _N
    idx_ref[...] = jnp.where(valid, v * _W + u, 0)
    re_ref[...] = jnp.where(valid, re * scale, 0.0)
    im_ref[...] = jnp.where(valid, im * scale, 0.0)


def _in_map(i):
    return (0, i)


def _in_map2(i):
    return (i, 0)


_tc_call = pl.pallas_call(
    _tc_body,
    grid=(_GRID,),
    in_specs=[
        pl.BlockSpec(memory_space=pltpu.SMEM),
        pl.BlockSpec(memory_space=pltpu.SMEM),
        pl.BlockSpec((3, _BLKC), _in_map),
        pl.BlockSpec((6, _BLKC), _in_map),
        pl.BlockSpec((2, _BLKC), _in_map),
        pl.BlockSpec((1, _BLKC), _in_map),
        pl.BlockSpec((1, _BLKC), _in_map),
    ],
    out_specs=[
        pl.BlockSpec((_BLK, _OC), lambda i: (i, 0)),
        pl.BlockSpec((_BLK, _OC), lambda i: (i, 0)),
        pl.BlockSpec((_BLK, _OC), lambda i: (i, 0)),
    ],
    out_shape=[
        jax.ShapeDtypeStruct((_PAD_ROWS, _OC), jnp.int32),
        jax.ShapeDtypeStruct((_PAD_ROWS, _OC), jnp.float32),
        jax.ShapeDtypeStruct((_PAD_ROWS, _OC), jnp.float32),
    ],
    compiler_params=pltpu.CompilerParams(
        allow_input_fusion=[False, False, True, True, True, True, True]),
)


def _sc_body(idx_hbm, re_hbm, im_hbm, out_hbm,
             idx_v, re_v, im_v, plane_v, rows_v, accum_a, accum_b,
             sem_i, sem_r, sem_m):
    c = lax.axis_index("c")
    s = lax.axis_index("s")
    wid = c * 16 + s
    base = wid * _CROWS

    cp_i = pltpu.async_copy(idx_hbm.at[pl.ds(base, _CROWS)], idx_v, sem_i)
    cp_r = pltpu.async_copy(re_hbm.at[pl.ds(base, _CROWS)], re_v, sem_r)
    cp_m = pltpu.async_copy(im_hbm.at[pl.ds(base, _CROWS)], im_v, sem_m)

    zeros16f = jnp.zeros((16,), jnp.float32)

    def zero_plane():
        @plsc.parallel_loop(0, _PR * 8, unroll=8)
        def _(k):
            plane_v[k >> 3, pl.ds((k & 7) * 16, 16)] = zeros16f

    zero_plane()

    @plsc.parallel_loop(0, _PR // 16, unroll=4)
    def _(k):
        rows_v[pl.ds(k * 16, 16)] = lax.iota(jnp.int32, 16) + k * 16

    @pl.when(s == 0)
    def _():
        pltpu.sync_copy(plane_v, accum_a)
        pltpu.sync_copy(plane_v, accum_b)

    plsc.subcore_barrier()
    cp_i.wait()
    cp_r.wait()

    def scatter_pass(val_v):
        @plsc.parallel_loop(0, _CROWS * (_OC // 16), unroll=8)
        def _(t):
            r = t >> 5
            q = lax.bitwise_and(t, 31) * 16
            iv = idx_v[r, pl.ds(q, 16)]
            hi = lax.shift_right_logical(iv, 7)
            lo = lax.bitwise_and(iv, 127)
            rv = val_v[r, pl.ds(q, 16)]
            plsc.addupdate_scatter(plane_v, [hi, lo], rv)

    scatter_pass(re_v)
    pltpu.sync_copy(plane_v, accum_a.at[rows_v], add=True)

    zero_plane()
    cp_m.wait()
    scatter_pass(im_v)
    pltpu.sync_copy(plane_v, accum_b.at[rows_v], add=True)

    plsc.subcore_barrier()

    orow = s * (_PR // 16)
    pltpu.sync_copy(accum_a.at[pl.ds(orow, _PR // 16)],
                    out_hbm.at[c, 0, pl.ds(orow, _PR // 16)])
    pltpu.sync_copy(accum_b.at[pl.ds(orow, _PR // 16)],
                    out_hbm.at[c, 1, pl.ds(orow, _PR // 16)])


_sc_call_cache = []


def _sc_call(idx, re, im):
    if not _sc_call_cache:
        _sc_call_cache.append(pl.kernel(
            _sc_body,
            out_type=jax.ShapeDtypeStruct((2, 2, _PR, _PC), jnp.float32),
            mesh=plsc.VectorSubcoreMesh(core_axis_name="c",
                                        subcore_axis_name="s"),
            compiler_params=pltpu.CompilerParams(needs_layout_passes=False),
            scratch_types=[
                pltpu.VMEM((_CROWS, _OC), jnp.int32),
                pltpu.VMEM((_CROWS, _OC), jnp.float32),
                pltpu.VMEM((_CROWS, _OC), jnp.float32),
                pltpu.VMEM((_PR, _PC), jnp.float32),
                pltpu.VMEM((_PR,), jnp.int32),
                pltpu.VMEM_SHARED((_PR, _PC), jnp.float32),
                pltpu.VMEM_SHARED((_PR, _PC), jnp.float32),
                pltpu.SemaphoreType.DMA,
                pltpu.SemaphoreType.DMA,
                pltpu.SemaphoreType.DMA,
            ],
        ))
    return _sc_call_cache[0](idx, re, im)


def kernel(means_3d, cov3d_precomp, signal_precomp, attenuation, gaus_radii,
           rx_pos, tx_pos, bg):
    mt = means_3d.T
    ct = cov3d_precomp.T
    st = signal_precomp.T
    att = attenuation[None, :]
    rad = gaus_radii[None, :]
    idx, re, im = _tc_call(rx_pos, tx_pos, mt, ct, st, att, rad)
    partial = _sc_call(idx, re, im)
    planes = partial[0] + partial[1]
    img = planes.reshape(2, _H, _W).transpose(1, 2, 0)
    return img + bg[None, None, :]
